# baseline (speedup 1.0000x reference)
import functools

import jax
import jax.numpy as jnp
from jax import lax
from jax.experimental import pallas as pl
from jax.experimental.pallas import tpu as pltpu
from jax.experimental.pallas import tpu_sc as plsc

NC = 2
NS = 16
LANES = 16
DH = 128
CB = 128


def _mesh():
    return plsc.VectorSubcoreMesh(
        core_axis_name="c", subcore_axis_name="s", num_cores=NC,
        num_subcores=NS)


def _zero_rows(buf, nrows):
    z = jnp.zeros((LANES,), jnp.float32)

    def body(i, _):
        for j in range(DH // LANES):
            buf[i, pl.ds(j * LANES, LANES)] = z
        return 0

    lax.fori_loop(0, nrows, body, 0)


def _make_prep(Np, Ep, V):
    RPT = Np // NS
    RCH = RPT // CB
    ECHT = Ep // NS // CB
    EC0 = ECHT // 2

    def body(xg, dst3, emb2, h0, degp, dstv, xv, rows, sem, deg_sh):
        c = lax.axis_index("c")
        s = lax.axis_index("s")
        zf = jnp.zeros((LANES,), jnp.float32)
        of = jnp.ones((LANES,), jnp.float32)

        pltpu.sync_copy(dst3.at[s], dstv)
        pltpu.sync_copy(xg.at[c, s], xv)

        def zb(i, _):
            for j in range(DH // LANES):
                rows[i, pl.ds(j * LANES, LANES)] = zf
            return 0
        lax.fori_loop(0, CB, zb, 0)
        for k in range(RCH):
            pltpu.sync_copy(rows, deg_sh.at[pl.ds(s * RPT + k * CB, CB)])
        plsc.subcore_barrier()

        def ob(i, _):
            for j in range(DH // LANES):
                rows[i, pl.ds(j * LANES, LANES)] = of
            return 0
        lax.fori_loop(0, CB, ob, 0)
        lo = lax.select(c == 0, 0, EC0)
        hi = lax.select(c == 0, EC0, ECHT)

        def acc(j, _):
            pltpu.sync_copy(rows, deg_sh.at[dstv.at[j]], add=True)
            return 0
        lax.fori_loop(lo, hi, acc, 0)
        plsc.subcore_barrier()
        pltpu.sync_copy(deg_sh.at[pl.ds(s * RPT, RPT)],
                        degp.at[pl.ds(c * Np + s * RPT, RPT)])

        for k in range(RCH):
            pltpu.async_copy(emb2.at[xv.at[k]], rows, sem).wait()
            pltpu.sync_copy(
                rows, h0.at[pl.ds(c * Np + s * RPT + k * CB, CB)])

    mesh = _mesh()
    out_type = (
        jax.ShapeDtypeStruct((NC * Np, DH), jnp.float32),
        jax.ShapeDtypeStruct((NC * Np, DH), jnp.float32),
    )
    scratch = [
        pltpu.VMEM((ECHT, CB), jnp.int32),
        pltpu.VMEM((RCH, CB), jnp.int32),
        pltpu.VMEM((CB, DH), jnp.float32),
        pltpu.SemaphoreType.DMA,
        pltpu.VMEM_SHARED((Np, DH), jnp.float32),
    ]
    return pl.kernel(body, out_type=out_type, mesh=mesh,
                     scratch_types=scratch,
                     compiler_params=pltpu.CompilerParams(
                         needs_layout_passes=False))


CBR = 128


def _make_spmm(Np, Ep):
    RPT = Np // NS
    CHT = Ep // NS // CBR

    def body(g_h, srcg, dst3, s_h, srcv, dstv, rows, sem, s_sh):
        c = lax.axis_index("c")
        s = lax.axis_index("s")

        _zero_rows(rows, CBR)
        for k in range(RPT // CBR):
            pltpu.sync_copy(rows, s_sh.at[pl.ds(s * RPT + k * CBR, CBR)])
        plsc.subcore_barrier()

        pltpu.sync_copy(srcg.at[c, s], srcv)
        pltpu.sync_copy(dst3.at[s], dstv)

        def chunk(j, _):
            pltpu.async_copy(g_h.at[srcv.at[j]], rows, sem).wait()
            pltpu.sync_copy(rows, s_sh.at[dstv.at[j]], add=True)
            return 0
        lax.fori_loop(0, CHT, chunk, 0)
        plsc.subcore_barrier()

        for k in range(RPT // CBR):
            pltpu.sync_copy(
                s_sh.at[pl.ds(s * RPT + k * CBR, CBR)],
                s_h.at[pl.ds(c * Np + s * RPT + k * CBR, CBR)])

    mesh = _mesh()
    out_type = jax.ShapeDtypeStruct((NC * Np, DH), jnp.float32)
    scratch = [
        pltpu.VMEM((CHT, CBR), jnp.int32),
        pltpu.VMEM((CHT, CBR), jnp.int32),
        pltpu.VMEM((CBR, DH), jnp.float32),
        pltpu.SemaphoreType.DMA,
        pltpu.VMEM_SHARED((Np, DH), jnp.float32),
    ]
    return pl.kernel(body, out_type=out_type, mesh=mesh,
                     scratch_types=scratch,
                     compiler_params=pltpu.CompilerParams(
                         needs_layout_passes=False))


def _scale_body(h0_ref, degp_ref, g0_ref, dinv_ref):
    dv = lax.rsqrt(degp_ref[0, :, :1] + degp_ref[1, :, :1] + 1.0)
    g0_ref[...] = h0_ref[...] * dv[None]
    dinv_ref[...] = dv


def _make_scale(Np, BM):
    grid = (Np // BM,)
    return pl.pallas_call(
        _scale_body,
        grid=grid,
        in_specs=[
            pl.BlockSpec((NC, BM, DH), lambda i: (0, i, 0)),
            pl.BlockSpec((NC, BM, DH), lambda i: (0, i, 0)),
        ],
        out_specs=[
            pl.BlockSpec((NC, BM, DH), lambda i: (0, i, 0)),
            pl.BlockSpec((BM, 1), lambda i: (i, 0)),
        ],
        out_shape=[
            jax.ShapeDtypeStruct((NC, Np, DH), jnp.float32),
            jax.ShapeDtypeStruct((Np, 1), jnp.float32),
        ],
    )


def _mm_body(last, s_ref, g_ref, dv_ref, w_ref, b_ref, o_ref):
    dv = dv_ref[...]
    a0 = (s_ref[0] + g_ref[0]) * dv
    a1 = (s_ref[1] + g_ref[1]) * dv
    w = w_ref[...]
    acc = (jnp.dot(a0, w[:DH], preferred_element_type=jnp.float32)
           + jnp.dot(a1, w[DH:], preferred_element_type=jnp.float32)
           + b_ref[...])
    if last:
        o_ref[...] = acc
    else:
        o_ref[0] = jnp.maximum(acc, 0.0) * dv


def _make_matmul(Np, BM, last):
    grid = (Np // BM, NC)
    in_specs = [
        pl.BlockSpec((NC, BM, DH), lambda i, j: (0, i, 0)),
        pl.BlockSpec((NC, BM, DH), lambda i, j: (0, i, 0)),
        pl.BlockSpec((BM, 1), lambda i, j: (i, 0)),
        pl.BlockSpec((NC * DH, DH), lambda i, j: (0, j)),
        pl.BlockSpec((1, DH), lambda i, j: (0, j)),
    ]
    if last:
        out_spec = pl.BlockSpec((BM, DH), lambda i, j: (i, j))
        out_shape = jax.ShapeDtypeStruct((Np, NC * DH), jnp.float32)
    else:
        out_spec = pl.BlockSpec((1, BM, DH), lambda i, j: (j, i, 0))
        out_shape = jax.ShapeDtypeStruct((NC, Np, DH), jnp.float32)
    return pl.pallas_call(
        functools.partial(_mm_body, last),
        grid=grid,
        in_specs=in_specs,
        out_specs=out_spec,
        out_shape=out_shape,
        compiler_params=pltpu.CompilerParams(
            dimension_semantics=("parallel", "parallel")),
    )


def kernel(x, edge_index, emb, Ws, bs):
    N = x.shape[0]
    V, D = emb.shape
    L = Ws.shape[0]
    E = edge_index.shape[1]
    assert D == NC * DH

    unit = NS * CB
    unit_e = NS * CBR * 4
    Np = ((N + unit - 1) // unit) * unit
    Ep = ((E + unit_e - 1) // unit_e) * unit_e

    x = x.astype(jnp.int32)
    src = edge_index[0].astype(jnp.int32)
    dst = edge_index[1].astype(jnp.int32)

    x_p = jnp.concatenate([x, jnp.zeros((Np - N,), jnp.int32)])
    xg = jnp.stack([x_p, x_p + V]).reshape(NC, NS, Np // NS // CB, CB)
    src_p = jnp.concatenate([src, jnp.full((Ep - E,), N, jnp.int32)])
    dst_p = jnp.concatenate([dst, jnp.full((Ep - E,), N, jnp.int32)])
    srcg = jnp.stack([src_p, src_p + Np]).reshape(
        NC, NS, Ep // NS // CBR, CBR)
    dst3p = dst_p.reshape(NS, Ep // NS // CB, CB)
    dst3s = dst_p.reshape(NS, Ep // NS // CBR, CBR)
    emb2 = emb.reshape(V, NC, DH).transpose(1, 0, 2).reshape(NC * V, DH)

    prep = _make_prep(Np, Ep, V)
    h0_flat, degp = prep(xg, dst3p, emb2)

    BM = 512
    scale = _make_scale(Np, BM)
    g, dinv = scale(h0_flat.reshape(NC, Np, DH), degp.reshape(NC, Np, DH))

    spmm = _make_spmm(Np, Ep)
    for i in range(L):
        s_flat = spmm(g.reshape(NC * Np, DH), srcg, dst3s)
        mm = _make_matmul(Np, BM, last=(i == L - 1))
        g = mm(s_flat.reshape(NC, Np, DH), g, dinv,
               Ws[i], bs[i].reshape(1, D))
    return g[:N]


# speedup vs baseline: 1.7199x; 1.7199x over previous
import functools

import jax
import jax.numpy as jnp
from jax import lax
from jax.experimental import pallas as pl
from jax.experimental.pallas import tpu as pltpu
from jax.experimental.pallas import tpu_sc as plsc

NC = 2
NS = 16
LANES = 16
DH = 128
CB = 128


def _mesh():
    return plsc.VectorSubcoreMesh(
        core_axis_name="c", subcore_axis_name="s", num_cores=NC,
        num_subcores=NS)


def _zero_rows(buf, nrows):
    z = jnp.zeros((LANES,), jnp.float32)

    def body(i, _):
        for j in range(DH // LANES):
            buf[i, pl.ds(j * LANES, LANES)] = z
        return 0

    lax.fori_loop(0, nrows, body, 0)


def _make_prep(Np, Ep, V):
    RPT = Np // NS
    RCH = RPT // CB
    ECHT = Ep // NS // CB
    EC0 = ECHT // 2

    def body(xg, dst3, emb2, h0, degp, dstv, xv, rows, sem, deg_sh):
        c = lax.axis_index("c")
        s = lax.axis_index("s")
        zf = jnp.zeros((LANES,), jnp.float32)
        of = jnp.ones((LANES,), jnp.float32)

        pltpu.sync_copy(dst3.at[s], dstv)
        pltpu.sync_copy(xg.at[c, s], xv)

        def zb(i, _):
            for j in range(DH // LANES):
                rows[i, pl.ds(j * LANES, LANES)] = zf
            return 0
        lax.fori_loop(0, CB, zb, 0)
        for k in range(RCH):
            pltpu.sync_copy(rows, deg_sh.at[pl.ds(s * RPT + k * CB, CB)])
        plsc.subcore_barrier()

        def ob(i, _):
            for j in range(DH // LANES):
                rows[i, pl.ds(j * LANES, LANES)] = of
            return 0
        lax.fori_loop(0, CB, ob, 0)
        lo = lax.select(c == 0, 0, EC0)
        hi = lax.select(c == 0, EC0, ECHT)

        def acc(j, _):
            pltpu.sync_copy(rows, deg_sh.at[dstv.at[j]], add=True)
            return 0
        lax.fori_loop(lo, hi, acc, 0)
        plsc.subcore_barrier()
        pltpu.sync_copy(deg_sh.at[pl.ds(s * RPT, RPT)],
                        degp.at[pl.ds(c * Np + s * RPT, RPT)])

        for k in range(RCH):
            pltpu.async_copy(emb2.at[xv.at[k]], rows, sem).wait()
            pltpu.sync_copy(
                rows, h0.at[pl.ds(c * Np + s * RPT + k * CB, CB)])

    mesh = _mesh()
    out_type = (
        jax.ShapeDtypeStruct((NC * Np, DH), jnp.float32),
        jax.ShapeDtypeStruct((NC * Np, DH), jnp.float32),
    )
    scratch = [
        pltpu.VMEM((ECHT, CB), jnp.int32),
        pltpu.VMEM((RCH, CB), jnp.int32),
        pltpu.VMEM((CB, DH), jnp.float32),
        pltpu.SemaphoreType.DMA,
        pltpu.VMEM_SHARED((Np, DH), jnp.float32),
    ]
    return pl.kernel(body, out_type=out_type, mesh=mesh,
                     scratch_types=scratch,
                     compiler_params=pltpu.CompilerParams(
                         needs_layout_passes=False))


CBR = 128


def _make_spmm(Np, Ep):
    RPT = Np // NS
    CHT = Ep // NS // CBR

    def body(g_h, srcg, dst3, s_h, srcv, dstv, rows, sem, s_sh):
        c = lax.axis_index("c")
        s = lax.axis_index("s")

        _zero_rows(rows, CBR)
        for k in range(RPT // CBR):
            pltpu.sync_copy(rows, s_sh.at[pl.ds(s * RPT + k * CBR, CBR)])
        plsc.subcore_barrier()

        pltpu.sync_copy(srcg.at[c, s], srcv)
        pltpu.sync_copy(dst3.at[s], dstv)

        def chunk(j, _):
            pltpu.async_copy(g_h.at[srcv.at[j]], rows, sem).wait()
            pltpu.sync_copy(rows, s_sh.at[dstv.at[j]], add=True)
            return 0
        lax.fori_loop(0, CHT, chunk, 0)
        plsc.subcore_barrier()

        for k in range(RPT // CBR):
            pltpu.sync_copy(
                s_sh.at[pl.ds(s * RPT + k * CBR, CBR)],
                s_h.at[pl.ds(c * Np + s * RPT + k * CBR, CBR)])

    mesh = _mesh()
    out_type = jax.ShapeDtypeStruct((NC * Np, DH), jnp.float32)
    scratch = [
        pltpu.VMEM((CHT, CBR), jnp.int32),
        pltpu.VMEM((CHT, CBR), jnp.int32),
        pltpu.VMEM((CBR, DH), jnp.float32),
        pltpu.SemaphoreType.DMA,
        pltpu.VMEM_SHARED((Np, DH), jnp.float32),
    ]
    return pl.kernel(body, out_type=out_type, mesh=mesh,
                     scratch_types=scratch,
                     compiler_params=pltpu.CompilerParams(
                         needs_layout_passes=False))


def _scale_body(h0_ref, degp_ref, g0_ref, dinv_ref):
    dv = lax.rsqrt(degp_ref[0, :, :1] + degp_ref[1, :, :1] + 1.0)
    g0_ref[...] = h0_ref[...] * dv[None]
    dinv_ref[...] = dv


def _make_scale(Np, BM):
    grid = (Np // BM,)
    return pl.pallas_call(
        _scale_body,
        grid=grid,
        in_specs=[
            pl.BlockSpec((NC, BM, DH), lambda i: (0, i, 0)),
            pl.BlockSpec((NC, BM, DH), lambda i: (0, i, 0)),
        ],
        out_specs=[
            pl.BlockSpec((NC, BM, DH), lambda i: (0, i, 0)),
            pl.BlockSpec((BM, 1), lambda i: (i, 0)),
        ],
        out_shape=[
            jax.ShapeDtypeStruct((NC, Np, DH), jnp.float32),
            jax.ShapeDtypeStruct((Np, 1), jnp.float32),
        ],
    )


def _mm_body(last, s_ref, g_ref, dv_ref, w_ref, b_ref, o_ref):
    dv = dv_ref[...]
    a0 = (s_ref[0] + g_ref[0]) * dv
    a1 = (s_ref[1] + g_ref[1]) * dv
    w = w_ref[...]
    acc = (jnp.dot(a0, w[:DH], preferred_element_type=jnp.float32)
           + jnp.dot(a1, w[DH:], preferred_element_type=jnp.float32)
           + b_ref[...])
    if last:
        o_ref[...] = acc
    else:
        o_ref[0] = jnp.maximum(acc, 0.0) * dv


def _make_matmul(Np, BM, last):
    grid = (Np // BM, NC)
    in_specs = [
        pl.BlockSpec((NC, BM, DH), lambda i, j: (0, i, 0)),
        pl.BlockSpec((NC, BM, DH), lambda i, j: (0, i, 0)),
        pl.BlockSpec((BM, 1), lambda i, j: (i, 0)),
        pl.BlockSpec((NC * DH, DH), lambda i, j: (0, j)),
        pl.BlockSpec((1, DH), lambda i, j: (0, j)),
    ]
    if last:
        out_spec = pl.BlockSpec((BM, DH), lambda i, j: (i, j))
        out_shape = jax.ShapeDtypeStruct((Np, NC * DH), jnp.float32)
    else:
        out_spec = pl.BlockSpec((1, BM, DH), lambda i, j: (j, i, 0))
        out_shape = jax.ShapeDtypeStruct((NC, Np, DH), jnp.float32)
    return pl.pallas_call(
        functools.partial(_mm_body, last),
        grid=grid,
        in_specs=in_specs,
        out_specs=out_spec,
        out_shape=out_shape,
        compiler_params=pltpu.CompilerParams(
            dimension_semantics=("parallel", "parallel")),
    )


def kernel(x, edge_index, emb, Ws, bs):
    N = x.shape[0]
    V, D = emb.shape
    L = Ws.shape[0]
    E = edge_index.shape[1]
    assert D == NC * DH

    unit = NS * CB
    Np = ((N + unit - 1) // unit) * unit
    Ep = ((E + unit - 1) // unit) * unit

    x = x.astype(jnp.int32)
    src = edge_index[0].astype(jnp.int32)
    dst = edge_index[1].astype(jnp.int32)

    x_p = jnp.concatenate([x, jnp.zeros((Np - N,), jnp.int32)])
    xg = jnp.stack([x_p, x_p + V]).reshape(NC, NS, Np // NS // CB, CB)
    pad_rows = N + jnp.arange(Ep - E, dtype=jnp.int32) % (Np - N)
    src_p = jnp.concatenate([src, pad_rows])
    dst_p = jnp.concatenate([dst, pad_rows])
    srcg = jnp.stack([src_p, src_p + Np]).reshape(
        NC, NS, Ep // NS // CBR, CBR)
    dst3p = dst_p.reshape(NS, Ep // NS // CB, CB)
    dst3s = dst_p.reshape(NS, Ep // NS // CBR, CBR)
    emb2 = emb.reshape(V, NC, DH).transpose(1, 0, 2).reshape(NC * V, DH)

    prep = _make_prep(Np, Ep, V)
    h0_flat, degp = prep(xg, dst3p, emb2)

    BM = 512
    scale = _make_scale(Np, BM)
    g, dinv = scale(h0_flat.reshape(NC, Np, DH), degp.reshape(NC, Np, DH))

    spmm = _make_spmm(Np, Ep)
    for i in range(L):
        s_flat = spmm(g.reshape(NC * Np, DH), srcg, dst3s)
        mm = _make_matmul(Np, BM, last=(i == L - 1))
        g = mm(s_flat.reshape(NC, Np, DH), g, dinv,
               Ws[i], bs[i].reshape(1, D))
    return g[:N]


# speedup vs baseline: 2.0889x; 1.2145x over previous
import functools

import jax
import jax.numpy as jnp
from jax import lax
from jax.experimental import pallas as pl
from jax.experimental.pallas import tpu as pltpu
from jax.experimental.pallas import tpu_sc as plsc

NC = 2
NS = 16
LANES = 16
DH = 128
CB = 128


def _mesh():
    return plsc.VectorSubcoreMesh(
        core_axis_name="c", subcore_axis_name="s", num_cores=NC,
        num_subcores=NS)


def _zero_rows(buf, nrows):
    z = jnp.zeros((LANES,), jnp.float32)

    def body(i, _):
        for j in range(DH // LANES):
            buf[i, pl.ds(j * LANES, LANES)] = z
        return 0

    lax.fori_loop(0, nrows, body, 0)


def _make_prep(Np, Ep, V):
    RPT = Np // NS
    RCH = RPT // CB
    ECHT = Ep // NS // CB
    EC0 = ECHT // 2

    def body(xg, dst3, emb2, h0, degp, dstv, xv, rows, sem, deg_sh):
        c = lax.axis_index("c")
        s = lax.axis_index("s")
        zf = jnp.zeros((LANES,), jnp.float32)
        of = jnp.ones((LANES,), jnp.float32)

        pltpu.sync_copy(dst3.at[s], dstv)
        pltpu.sync_copy(xg.at[c, s], xv)

        def zb(i, _):
            for j in range(DH // LANES):
                rows[i, pl.ds(j * LANES, LANES)] = zf
            return 0
        lax.fori_loop(0, CB, zb, 0)
        for k in range(RCH):
            pltpu.sync_copy(rows, deg_sh.at[pl.ds(s * RPT + k * CB, CB)])
        plsc.subcore_barrier()

        def ob(i, _):
            for j in range(DH // LANES):
                rows[i, pl.ds(j * LANES, LANES)] = of
            return 0
        lax.fori_loop(0, CB, ob, 0)
        lo = lax.select(c == 0, 0, EC0)
        hi = lax.select(c == 0, EC0, ECHT)

        def acc(j, _):
            pltpu.sync_copy(rows, deg_sh.at[dstv.at[j]], add=True)
            return 0
        lax.fori_loop(lo, hi, acc, 0)
        plsc.subcore_barrier()
        pltpu.sync_copy(deg_sh.at[pl.ds(s * RPT, RPT)],
                        degp.at[pl.ds(c * Np + s * RPT, RPT)])

        for k in range(RCH):
            pltpu.async_copy(emb2.at[xv.at[k]], rows, sem).wait()
            pltpu.sync_copy(
                rows, h0.at[pl.ds(c * Np + s * RPT + k * CB, CB)])

    mesh = _mesh()
    out_type = (
        jax.ShapeDtypeStruct((NC * Np, DH), jnp.float32),
        jax.ShapeDtypeStruct((NC * Np, DH), jnp.float32),
    )
    scratch = [
        pltpu.VMEM((ECHT, CB), jnp.int32),
        pltpu.VMEM((RCH, CB), jnp.int32),
        pltpu.VMEM((CB, DH), jnp.float32),
        pltpu.SemaphoreType.DMA,
        pltpu.VMEM_SHARED((Np, DH), jnp.float32),
    ]
    return pl.kernel(body, out_type=out_type, mesh=mesh,
                     scratch_types=scratch,
                     compiler_params=pltpu.CompilerParams(
                         needs_layout_passes=False))


CBR = 128


def _make_spmm(Np, Ep):
    RPT = Np // NS
    CHT = Ep // NS // CBR
    CHT2 = CHT // 2

    def body(g_h, srcg, dst3, s_h, srcv, dstv, r0, r1,
             gs0, gs1, ss0, ss1, s_sh):
        c = lax.axis_index("c")
        s = lax.axis_index("s")
        rows = [r0, r1]
        gsem = [gs0, gs1]
        ssem = [ss0, ss1]

        _zero_rows(r0, CBR)
        for k in range(RPT // CBR):
            pltpu.sync_copy(r0, s_sh.at[pl.ds(s * RPT + k * CBR, CBR)])
        plsc.subcore_barrier()

        def wait_gather(p):
            pltpu.make_async_copy(g_h.at[srcv.at[0]], rows[p],
                                  gsem[p]).wait()

        def wait_scatter(p):
            pltpu.make_async_copy(rows[p], s_sh.at[dstv.at[0]],
                                  ssem[p]).wait()

        for h in range(2):
            if h > 0:
                wait_scatter(0)
                wait_scatter(1)
            pltpu.sync_copy(srcg.at[c, s, h], srcv)
            pltpu.sync_copy(dst3.at[s, h], dstv)

            def step(i, _):
                for b in range(2):
                    j = 2 * i + b

                    @pl.when(j >= 2)
                    def _():
                        wait_scatter(b)
                    pltpu.async_copy(g_h.at[srcv.at[j]], rows[b], gsem[b])
                    wait_gather(b)
                    pltpu.async_copy(rows[b], s_sh.at[dstv.at[j]],
                                     ssem[b], add=True)
                return 0
            lax.fori_loop(0, CHT2 // 2, step, 0)
        wait_scatter(0)
        wait_scatter(1)
        plsc.subcore_barrier()

        for k in range(RPT // CBR):
            pltpu.sync_copy(
                s_sh.at[pl.ds(s * RPT + k * CBR, CBR)],
                s_h.at[pl.ds(c * Np + s * RPT + k * CBR, CBR)])

    mesh = _mesh()
    out_type = jax.ShapeDtypeStruct((NC * Np, DH), jnp.float32)
    scratch = (
        [pltpu.VMEM((CHT2, CBR), jnp.int32)] * 2
        + [pltpu.VMEM((CBR, DH), jnp.float32)] * 2
        + [pltpu.SemaphoreType.DMA] * 4
        + [pltpu.VMEM_SHARED((Np, DH), jnp.float32)]
    )
    return pl.kernel(body, out_type=out_type, mesh=mesh,
                     scratch_types=scratch,
                     compiler_params=pltpu.CompilerParams(
                         needs_layout_passes=False))


def _scale_body(h0_ref, degp_ref, g0_ref, dinv_ref):
    dv = lax.rsqrt(degp_ref[0, :, :1] + degp_ref[1, :, :1] + 1.0)
    g0_ref[...] = h0_ref[...] * dv[None]
    dinv_ref[...] = dv


def _make_scale(Np, BM):
    grid = (Np // BM,)
    return pl.pallas_call(
        _scale_body,
        grid=grid,
        in_specs=[
            pl.BlockSpec((NC, BM, DH), lambda i: (0, i, 0)),
            pl.BlockSpec((NC, BM, DH), lambda i: (0, i, 0)),
        ],
        out_specs=[
            pl.BlockSpec((NC, BM, DH), lambda i: (0, i, 0)),
            pl.BlockSpec((BM, 1), lambda i: (i, 0)),
        ],
        out_shape=[
            jax.ShapeDtypeStruct((NC, Np, DH), jnp.float32),
            jax.ShapeDtypeStruct((Np, 1), jnp.float32),
        ],
    )


def _mm_body(last, s_ref, g_ref, dv_ref, w_ref, b_ref, o_ref):
    dv = dv_ref[...]
    a0 = (s_ref[0] + g_ref[0]) * dv
    a1 = (s_ref[1] + g_ref[1]) * dv
    w = w_ref[...]
    acc = (jnp.dot(a0, w[:DH], preferred_element_type=jnp.float32)
           + jnp.dot(a1, w[DH:], preferred_element_type=jnp.float32)
           + b_ref[...])
    if last:
        o_ref[...] = acc
    else:
        o_ref[0] = jnp.maximum(acc, 0.0) * dv


def _make_matmul(Np, BM, last):
    grid = (Np // BM, NC)
    in_specs = [
        pl.BlockSpec((NC, BM, DH), lambda i, j: (0, i, 0)),
        pl.BlockSpec((NC, BM, DH), lambda i, j: (0, i, 0)),
        pl.BlockSpec((BM, 1), lambda i, j: (i, 0)),
        pl.BlockSpec((NC * DH, DH), lambda i, j: (0, j)),
        pl.BlockSpec((1, DH), lambda i, j: (0, j)),
    ]
    if last:
        out_spec = pl.BlockSpec((BM, DH), lambda i, j: (i, j))
        out_shape = jax.ShapeDtypeStruct((Np, NC * DH), jnp.float32)
    else:
        out_spec = pl.BlockSpec((1, BM, DH), lambda i, j: (j, i, 0))
        out_shape = jax.ShapeDtypeStruct((NC, Np, DH), jnp.float32)
    return pl.pallas_call(
        functools.partial(_mm_body, last),
        grid=grid,
        in_specs=in_specs,
        out_specs=out_spec,
        out_shape=out_shape,
        compiler_params=pltpu.CompilerParams(
            dimension_semantics=("parallel", "parallel")),
    )


def kernel(x, edge_index, emb, Ws, bs):
    N = x.shape[0]
    V, D = emb.shape
    L = Ws.shape[0]
    E = edge_index.shape[1]
    assert D == NC * DH

    unit = NS * CB
    unit_e = NS * CBR * 4
    Np = ((N + unit - 1) // unit) * unit
    Ep = ((E + unit_e - 1) // unit_e) * unit_e

    x = x.astype(jnp.int32)
    src = edge_index[0].astype(jnp.int32)
    dst = edge_index[1].astype(jnp.int32)

    x_p = jnp.concatenate([x, jnp.zeros((Np - N,), jnp.int32)])
    xg = jnp.stack([x_p, x_p + V]).reshape(NC, NS, Np // NS // CB, CB)
    pad_rows = N + jnp.arange(Ep - E, dtype=jnp.int32) % (Np - N)
    src_p = jnp.concatenate([src, pad_rows])
    dst_p = jnp.concatenate([dst, pad_rows])
    cht2 = Ep // NS // CBR // 2
    srcg = jnp.stack([src_p, src_p + Np]).reshape(NC, NS, 2, cht2, CBR)
    dst3p = dst_p.reshape(NS, Ep // NS // CB, CB)
    dst3s = dst_p.reshape(NS, 2, cht2, CBR)
    emb2 = emb.reshape(V, NC, DH).transpose(1, 0, 2).reshape(NC * V, DH)

    prep = _make_prep(Np, Ep, V)
    h0_flat, degp = prep(xg, dst3p, emb2)

    BM = 512
    scale = _make_scale(Np, BM)
    g, dinv = scale(h0_flat.reshape(NC, Np, DH), degp.reshape(NC, Np, DH))

    spmm = _make_spmm(Np, Ep)
    for i in range(L):
        s_flat = spmm(g.reshape(NC * Np, DH), srcg, dst3s)
        mm = _make_matmul(Np, BM, last=(i == L - 1))
        g = mm(s_flat.reshape(NC, Np, DH), g, dinv,
               Ws[i], bs[i].reshape(1, D))
    return g[:N]


# speedup vs baseline: 2.2985x; 1.1003x over previous
import functools

import jax
import jax.numpy as jnp
from jax import lax
from jax.experimental import pallas as pl
from jax.experimental.pallas import tpu as pltpu
from jax.experimental.pallas import tpu_sc as plsc

NC = 2
NS = 16
LANES = 16
DH = 128
CB = 128


def _mesh():
    return plsc.VectorSubcoreMesh(
        core_axis_name="c", subcore_axis_name="s", num_cores=NC,
        num_subcores=NS)


def _zero_rows(buf, nrows):
    z = jnp.zeros((LANES,), jnp.float32)

    def body(i, _):
        for j in range(DH // LANES):
            buf[i, pl.ds(j * LANES, LANES)] = z
        return 0

    lax.fori_loop(0, nrows, body, 0)


def _make_prep(Np, Ep, V):
    RPT = Np // NS
    RCH = RPT // CB
    ECHT = Ep // NS // CB
    EC0 = ECHT // 2

    def body(xg, dst3, emb2, h0, degp, dstv, xv, rows, sem, deg_sh):
        c = lax.axis_index("c")
        s = lax.axis_index("s")
        zf = jnp.zeros((LANES,), jnp.float32)
        of = jnp.ones((LANES,), jnp.float32)

        pltpu.sync_copy(dst3.at[s], dstv)
        pltpu.sync_copy(xg.at[c, s], xv)

        def zb(i, _):
            for j in range(DH // LANES):
                rows[i, pl.ds(j * LANES, LANES)] = zf
            return 0
        lax.fori_loop(0, CB, zb, 0)
        for k in range(RCH):
            pltpu.sync_copy(rows, deg_sh.at[pl.ds(s * RPT + k * CB, CB)])
        plsc.subcore_barrier()

        def ob(i, _):
            for j in range(DH // LANES):
                rows[i, pl.ds(j * LANES, LANES)] = of
            return 0
        lax.fori_loop(0, CB, ob, 0)
        lo = lax.select(c == 0, 0, EC0)
        hi = lax.select(c == 0, EC0, ECHT)

        def acc(j, _):
            pltpu.sync_copy(rows, deg_sh.at[dstv.at[j]], add=True)
            return 0
        lax.fori_loop(lo, hi, acc, 0)
        plsc.subcore_barrier()
        pltpu.sync_copy(deg_sh.at[pl.ds(s * RPT, RPT)],
                        degp.at[pl.ds(c * Np + s * RPT, RPT)])

        for k in range(RCH):
            pltpu.async_copy(emb2.at[xv.at[k]], rows, sem).wait()
            pltpu.sync_copy(
                rows, h0.at[pl.ds(c * Np + s * RPT + k * CB, CB)])

    mesh = _mesh()
    out_type = (
        jax.ShapeDtypeStruct((NC * Np, DH), jnp.float32),
        jax.ShapeDtypeStruct((NC * Np, DH), jnp.float32),
    )
    scratch = [
        pltpu.VMEM((ECHT, CB), jnp.int32),
        pltpu.VMEM((RCH, CB), jnp.int32),
        pltpu.VMEM((CB, DH), jnp.float32),
        pltpu.SemaphoreType.DMA,
        pltpu.VMEM_SHARED((Np, DH), jnp.float32),
    ]
    return pl.kernel(body, out_type=out_type, mesh=mesh,
                     scratch_types=scratch,
                     compiler_params=pltpu.CompilerParams(
                         needs_layout_passes=False))


CBR = 128


def _make_spmm(Np, Ep):
    RPT = Np // NS
    CHT = Ep // NS // CBR
    CHT2 = CHT // 2

    def body(g_h, srcg, dst3, s_h, srcv, dstv, r0, r1,
             gs0, gs1, ss0, ss1, s_sh):
        c = lax.axis_index("c")
        s = lax.axis_index("s")
        rows = [r0, r1]
        gsem = [gs0, gs1]
        ssem = [ss0, ss1]

        _zero_rows(r0, CBR)
        for k in range(RPT // CBR):
            pltpu.sync_copy(r0, s_sh.at[pl.ds(s * RPT + k * CBR, CBR)])
        plsc.subcore_barrier()

        def wait_gather(p):
            pltpu.make_async_copy(g_h.at[srcv.at[0]], rows[p],
                                  gsem[p]).wait()

        def wait_scatter(p):
            pltpu.make_async_copy(rows[p], s_sh.at[dstv.at[0]],
                                  ssem[p]).wait()

        for h in range(2):
            if h > 0:
                wait_scatter(0)
                wait_scatter(1)
            pltpu.sync_copy(srcg.at[c, s, h], srcv)
            pltpu.sync_copy(dst3.at[s, h], dstv)

            def step(i, _):
                for b in range(2):
                    j = 2 * i + b

                    @pl.when(j >= 2)
                    def _():
                        wait_scatter(b)
                    pltpu.async_copy(g_h.at[srcv.at[j]], rows[b], gsem[b])
                    wait_gather(b)
                    pltpu.async_copy(rows[b], s_sh.at[dstv.at[j]],
                                     ssem[b], add=True)
                return 0
            lax.fori_loop(0, CHT2 // 2, step, 0)
        wait_scatter(0)
        wait_scatter(1)
        plsc.subcore_barrier()

        for k in range(RPT // CBR):
            pltpu.sync_copy(
                s_sh.at[pl.ds(s * RPT + k * CBR, CBR)],
                s_h.at[pl.ds(c * Np + s * RPT + k * CBR, CBR)])

    mesh = _mesh()
    out_type = jax.ShapeDtypeStruct((NC * Np, DH), jnp.float32)
    scratch = (
        [pltpu.VMEM((CHT2, CBR), jnp.int32)] * 2
        + [pltpu.VMEM((CBR, DH), jnp.float32)] * 2
        + [pltpu.SemaphoreType.DMA] * 4
        + [pltpu.VMEM_SHARED((Np, DH), jnp.float32)]
    )
    return pl.kernel(body, out_type=out_type, mesh=mesh,
                     scratch_types=scratch,
                     compiler_params=pltpu.CompilerParams(
                         needs_layout_passes=False))


def _scale_body(h0_ref, degp_ref, g0_ref, dinv_ref):
    dv = lax.rsqrt(degp_ref[0, :, :1] + degp_ref[1, :, :1] + 1.0)
    g0_ref[...] = h0_ref[...] * dv[None]
    dinv_ref[...] = dv


def _make_scale(Np, BM):
    grid = (Np // BM,)
    return pl.pallas_call(
        _scale_body,
        grid=grid,
        in_specs=[
            pl.BlockSpec((NC, BM, DH), lambda i: (0, i, 0)),
            pl.BlockSpec((NC, BM, DH), lambda i: (0, i, 0)),
        ],
        out_specs=[
            pl.BlockSpec((NC, BM, DH), lambda i: (0, i, 0)),
            pl.BlockSpec((BM, 1), lambda i: (i, 0)),
        ],
        out_shape=[
            jax.ShapeDtypeStruct((NC, Np, DH), jnp.float32),
            jax.ShapeDtypeStruct((Np, 1), jnp.float32),
        ],
    )


def _mm_body(last, s_ref, g_ref, dv_ref, w_ref, b_ref, o_ref):
    dv = dv_ref[...]
    a0 = (s_ref[0] + g_ref[0]) * dv
    a1 = (s_ref[1] + g_ref[1]) * dv
    w = w_ref[...]
    acc = (jnp.dot(a0, w[:DH], preferred_element_type=jnp.float32)
           + jnp.dot(a1, w[DH:], preferred_element_type=jnp.float32)
           + b_ref[...])
    if last:
        o_ref[...] = acc
    else:
        h = jnp.maximum(acc, 0.0) * dv
        o_ref[0] = h[:, :DH]
        o_ref[1] = h[:, DH:]


def _make_matmul(Np, BM, last):
    grid = (Np // BM,)
    in_specs = [
        pl.BlockSpec((NC, BM, DH), lambda i: (0, i, 0)),
        pl.BlockSpec((NC, BM, DH), lambda i: (0, i, 0)),
        pl.BlockSpec((BM, 1), lambda i: (i, 0)),
        pl.BlockSpec((NC * DH, NC * DH), lambda i: (0, 0)),
        pl.BlockSpec((1, NC * DH), lambda i: (0, 0)),
    ]
    if last:
        out_spec = pl.BlockSpec((BM, NC * DH), lambda i: (i, 0))
        out_shape = jax.ShapeDtypeStruct((Np, NC * DH), jnp.float32)
    else:
        out_spec = pl.BlockSpec((NC, BM, DH), lambda i: (0, i, 0))
        out_shape = jax.ShapeDtypeStruct((NC, Np, DH), jnp.float32)
    return pl.pallas_call(
        functools.partial(_mm_body, last),
        grid=grid,
        in_specs=in_specs,
        out_specs=out_spec,
        out_shape=out_shape,
        compiler_params=pltpu.CompilerParams(
            dimension_semantics=("parallel",)),
    )


def kernel(x, edge_index, emb, Ws, bs):
    N = x.shape[0]
    V, D = emb.shape
    L = Ws.shape[0]
    E = edge_index.shape[1]
    assert D == NC * DH

    unit = NS * CB
    unit_e = NS * CBR * 4
    Np = ((N + unit - 1) // unit) * unit
    Ep = ((E + unit_e - 1) // unit_e) * unit_e

    x = x.astype(jnp.int32)
    src = edge_index[0].astype(jnp.int32)
    dst = edge_index[1].astype(jnp.int32)

    x_p = jnp.concatenate([x, jnp.zeros((Np - N,), jnp.int32)])
    xg = jnp.stack([x_p, x_p + V]).reshape(NC, NS, Np // NS // CB, CB)
    pad_rows = N + jnp.arange(Ep - E, dtype=jnp.int32) % (Np - N)
    src_p = jnp.concatenate([src, pad_rows])
    dst_p = jnp.concatenate([dst, pad_rows])
    cht2 = Ep // NS // CBR // 2
    srcg = jnp.stack([src_p, src_p + Np]).reshape(NC, NS, 2, cht2, CBR)
    dst3p = dst_p.reshape(NS, Ep // NS // CB, CB)
    dst3s = dst_p.reshape(NS, 2, cht2, CBR)
    emb2 = emb.reshape(V, NC, DH).transpose(1, 0, 2).reshape(NC * V, DH)

    prep = _make_prep(Np, Ep, V)
    h0_flat, degp = prep(xg, dst3p, emb2)

    BM = 512
    scale = _make_scale(Np, BM)
    g, dinv = scale(h0_flat.reshape(NC, Np, DH), degp.reshape(NC, Np, DH))

    spmm = _make_spmm(Np, Ep)
    for i in range(L):
        s_flat = spmm(g.reshape(NC * Np, DH), srcg, dst3s)
        mm = _make_matmul(Np, BM, last=(i == L - 1))
        g = mm(s_flat.reshape(NC, Np, DH), g, dinv,
               Ws[i], bs[i].reshape(1, D))
    return g[:N]


# speedup vs baseline: 2.3903x; 1.0399x over previous
import functools

import jax
import jax.numpy as jnp
from jax import lax
from jax.experimental import pallas as pl
from jax.experimental.pallas import tpu as pltpu
from jax.experimental.pallas import tpu_sc as plsc

NC = 2
NS = 16
LANES = 16
DH = 128
CB = 128


def _mesh():
    return plsc.VectorSubcoreMesh(
        core_axis_name="c", subcore_axis_name="s", num_cores=NC,
        num_subcores=NS)


def _zero_rows(buf, nrows):
    z = jnp.zeros((LANES,), jnp.float32)

    def body(i, _):
        for j in range(DH // LANES):
            buf[i, pl.ds(j * LANES, LANES)] = z
        return 0

    lax.fori_loop(0, nrows, body, 0)


def _make_prep(Np, Ep, V):
    RPT = Np // NS
    RCH = RPT // CB
    ECHT = Ep // NS // CB
    EC0 = ECHT // 2

    def body(xg, dst3, emb2, h0, degp, dstv, xv, rows, sem, deg_sh):
        c = lax.axis_index("c")
        s = lax.axis_index("s")
        zf = jnp.zeros((LANES,), jnp.float32)
        of = jnp.ones((LANES,), jnp.float32)

        pltpu.sync_copy(dst3.at[s], dstv)
        pltpu.sync_copy(xg.at[c, s], xv)

        def zb(i, _):
            for j in range(DH // LANES):
                rows[i, pl.ds(j * LANES, LANES)] = zf
            return 0
        lax.fori_loop(0, CB, zb, 0)
        for k in range(RCH):
            pltpu.sync_copy(rows, deg_sh.at[pl.ds(s * RPT + k * CB, CB)])
        plsc.subcore_barrier()

        def ob(i, _):
            for j in range(DH // LANES):
                rows[i, pl.ds(j * LANES, LANES)] = of
            return 0
        lax.fori_loop(0, CB, ob, 0)
        lo = lax.select(c == 0, 0, EC0)
        hi = lax.select(c == 0, EC0, ECHT)

        def acc(j, _):
            pltpu.sync_copy(rows, deg_sh.at[dstv.at[j]], add=True)
            return 0
        lax.fori_loop(lo, hi, acc, 0)
        plsc.subcore_barrier()
        pltpu.sync_copy(deg_sh.at[pl.ds(s * RPT, RPT)],
                        degp.at[pl.ds(c * Np + s * RPT, RPT)])

        for k in range(RCH):
            pltpu.async_copy(emb2.at[xv.at[k]], rows, sem).wait()
            pltpu.sync_copy(
                rows, h0.at[pl.ds(c * Np + s * RPT + k * CB, CB)])

    mesh = _mesh()
    out_type = (
        jax.ShapeDtypeStruct((NC * Np, DH), jnp.float32),
        jax.ShapeDtypeStruct((NC * Np, DH), jnp.float32),
    )
    scratch = [
        pltpu.VMEM((ECHT, CB), jnp.int32),
        pltpu.VMEM((RCH, CB), jnp.int32),
        pltpu.VMEM((CB, DH), jnp.float32),
        pltpu.SemaphoreType.DMA,
        pltpu.VMEM_SHARED((Np, DH), jnp.float32),
    ]
    return pl.kernel(body, out_type=out_type, mesh=mesh,
                     scratch_types=scratch,
                     compiler_params=pltpu.CompilerParams(
                         needs_layout_passes=False))


CBR = 128


def _make_spmm(Np, Ep):
    RPT = Np // NS
    CHT = Ep // NS // CBR
    CHT2 = CHT // 2

    def body(g_h, srcg, dst3, s_h, srcv, dstv, r0, r1,
             gs0, gs1, ss0, ss1, s_sh):
        c = lax.axis_index("c")
        s = lax.axis_index("s")
        rows = [r0, r1]
        gsem = [gs0, gs1]
        ssem = [ss0, ss1]

        _zero_rows(r0, CBR)
        for k in range(RPT // CBR):
            pltpu.sync_copy(r0, s_sh.at[pl.ds(s * RPT + k * CBR, CBR)])
        plsc.subcore_barrier()

        def wait_gather(p):
            pltpu.make_async_copy(g_h.at[srcv.at[0]], rows[p],
                                  gsem[p]).wait()

        def wait_scatter(p):
            pltpu.make_async_copy(rows[p], s_sh.at[dstv.at[0]],
                                  ssem[p]).wait()

        for h in range(2):
            if h > 0:
                wait_scatter(0)
                wait_scatter(1)
            pltpu.sync_copy(srcg.at[c, s, h], srcv)
            pltpu.sync_copy(dst3.at[s, h], dstv)

            def step(i, _):
                for b in range(2):
                    j = 2 * i + b

                    @pl.when(j >= 2)
                    def _():
                        wait_scatter(b)
                    pltpu.async_copy(g_h.at[srcv.at[j]], rows[b], gsem[b])
                    wait_gather(b)
                    pltpu.async_copy(rows[b], s_sh.at[dstv.at[j]],
                                     ssem[b], add=True)
                return 0
            lax.fori_loop(0, CHT2 // 2, step, 0)
        wait_scatter(0)
        wait_scatter(1)
        plsc.subcore_barrier()

        for k in range(RPT // CBR):
            pltpu.sync_copy(
                s_sh.at[pl.ds(s * RPT + k * CBR, CBR)],
                s_h.at[pl.ds(c * Np + s * RPT + k * CBR, CBR)])

    mesh = _mesh()
    out_type = jax.ShapeDtypeStruct((NC * Np, DH), jnp.float32)
    scratch = (
        [pltpu.VMEM((CHT2, CBR), jnp.int32)] * 2
        + [pltpu.VMEM((CBR, DH), jnp.float32)] * 2
        + [pltpu.SemaphoreType.DMA] * 4
        + [pltpu.VMEM_SHARED((Np, DH), jnp.float32)]
    )
    return pl.kernel(body, out_type=out_type, mesh=mesh,
                     scratch_types=scratch,
                     compiler_params=pltpu.CompilerParams(
                         needs_layout_passes=False))


def _scale_body(h0_ref, degp_ref, g0_ref, dinv_ref):
    dv = lax.rsqrt(degp_ref[0, :, :1] + degp_ref[1, :, :1] + 1.0)
    g0_ref[...] = h0_ref[...] * dv[None]
    dinv_ref[...] = dv


def _make_scale(Np, BM):
    grid = (Np // BM,)
    return pl.pallas_call(
        _scale_body,
        grid=grid,
        in_specs=[
            pl.BlockSpec((NC, BM, DH), lambda i: (0, i, 0)),
            pl.BlockSpec((NC, BM, DH), lambda i: (0, i, 0)),
        ],
        out_specs=[
            pl.BlockSpec((NC, BM, DH), lambda i: (0, i, 0)),
            pl.BlockSpec((BM, 1), lambda i: (i, 0)),
        ],
        out_shape=[
            jax.ShapeDtypeStruct((NC, Np, DH), jnp.float32),
            jax.ShapeDtypeStruct((Np, 1), jnp.float32),
        ],
    )


def _mm_body(last, s_ref, g_ref, dv_ref, w_ref, b_ref, o_ref):
    dv = dv_ref[...]
    a0 = (s_ref[0] + g_ref[0]) * dv
    a1 = (s_ref[1] + g_ref[1]) * dv
    w = w_ref[...]
    acc = (jnp.dot(a0, w[:DH], preferred_element_type=jnp.float32)
           + jnp.dot(a1, w[DH:], preferred_element_type=jnp.float32)
           + b_ref[...])
    if last:
        o_ref[...] = acc
    else:
        h = jnp.maximum(acc, 0.0) * dv
        o_ref[0] = h[:, :DH]
        o_ref[1] = h[:, DH:]


def _make_matmul(Np, BM, last):
    grid = (Np // BM,)
    in_specs = [
        pl.BlockSpec((NC, BM, DH), lambda i: (0, i, 0)),
        pl.BlockSpec((NC, BM, DH), lambda i: (0, i, 0)),
        pl.BlockSpec((BM, 1), lambda i: (i, 0)),
        pl.BlockSpec((NC * DH, NC * DH), lambda i: (0, 0)),
        pl.BlockSpec((1, NC * DH), lambda i: (0, 0)),
    ]
    if last:
        out_spec = pl.BlockSpec((BM, NC * DH), lambda i: (i, 0))
        out_shape = jax.ShapeDtypeStruct((Np, NC * DH), jnp.float32)
    else:
        out_spec = pl.BlockSpec((NC, BM, DH), lambda i: (0, i, 0))
        out_shape = jax.ShapeDtypeStruct((NC, Np, DH), jnp.float32)
    return pl.pallas_call(
        functools.partial(_mm_body, last),
        grid=grid,
        in_specs=in_specs,
        out_specs=out_spec,
        out_shape=out_shape,
        compiler_params=pltpu.CompilerParams(
            dimension_semantics=("parallel",)),
    )


def kernel(x, edge_index, emb, Ws, bs):
    N = x.shape[0]
    V, D = emb.shape
    L = Ws.shape[0]
    E = edge_index.shape[1]
    assert D == NC * DH

    unit = NS * CB
    unit_e = NS * CBR * 4
    Np = ((N + unit - 1) // unit) * unit
    Ep = ((E + unit_e - 1) // unit_e) * unit_e

    x = x.astype(jnp.int32)
    src = edge_index[0].astype(jnp.int32)
    dst = edge_index[1].astype(jnp.int32)

    x_p = jnp.concatenate([x, jnp.zeros((Np - N,), jnp.int32)])
    xg = jnp.stack([x_p, x_p + V]).reshape(NC, NS, Np // NS // CB, CB)
    pad_rows = N + jnp.arange(Ep - E, dtype=jnp.int32) % (Np - N)
    src_p = jnp.concatenate([src, pad_rows])
    dst_p = jnp.concatenate([dst, pad_rows])
    cht2 = Ep // NS // CBR // 2
    srcg = jnp.stack([src_p, src_p + Np]).reshape(NC, NS, 2, cht2, CBR)
    dst3p = dst_p.reshape(NS, Ep // NS // CB, CB)
    dst3s = dst_p.reshape(NS, 2, cht2, CBR)
    emb2 = emb.reshape(V, NC, DH).transpose(1, 0, 2).reshape(NC * V, DH)

    prep = _make_prep(Np, Ep, V)
    h0_flat, degp = prep(xg, dst3p, emb2)

    BM = 1024
    scale = _make_scale(Np, BM)
    g, dinv = scale(h0_flat.reshape(NC, Np, DH), degp.reshape(NC, Np, DH))

    spmm = _make_spmm(Np, Ep)
    for i in range(L):
        s_flat = spmm(g.reshape(NC * Np, DH), srcg, dst3s)
        mm = _make_matmul(Np, BM, last=(i == L - 1))
        g = mm(s_flat.reshape(NC, Np, DH), g, dinv,
               Ws[i], bs[i].reshape(1, D))
    return g[:N]
